# Initial kernel scaffold; baseline (speedup 1.0000x reference)
import jax
import jax.numpy as jnp
from jax.experimental import pallas as pl


def kernel(positive_item, entity_embedding, relation_embedding):
    raise NotImplementedError("write your pallas kernel here")


# speedup vs baseline: 1.1985x; 1.1985x over previous
import functools

import jax
import jax.numpy as jnp
from jax import lax
from jax.experimental import pallas as pl
from jax.experimental.pallas import tpu as pltpu
from jax.experimental.pallas import tpu_sc as plsc

D = 64
B = 16384
NC = 2
NS = 16
NW = NC * NS
BPW = B // NW
CHUNK = 128
NCH = BPW // CHUNK

_mesh = plsc.VectorSubcoreMesh(core_axis_name="c", subcore_axis_name="s")


@functools.partial(
    pl.kernel,
    mesh=_mesh,
    compiler_params=pltpu.CompilerParams(use_tc_tiling_on_sc=False),
    out_type=jax.ShapeDtypeStruct((B, 3 * D), jnp.float32),
    scratch_types=[
        pltpu.VMEM((3, NCH, CHUNK), jnp.int32),
        pltpu.VMEM((BPW, D), jnp.float32),
        pltpu.VMEM((BPW, D), jnp.float32),
        pltpu.VMEM((BPW, D), jnp.float32),
        pltpu.SemaphoreType.DMA,
        pltpu.SemaphoreType.DMA,
        pltpu.SemaphoreType.DMA,
    ],
)
def _gather_kernel(idx_hbm, ent_hbm, rel_hbm, out_hbm,
                   idx_v, head_v, relv_v, tail_v, s0, s1, s2):
    wid = lax.axis_index("s") * NC + lax.axis_index("c")
    base = wid * BPW
    pltpu.sync_copy(idx_hbm.at[wid], idx_v)
    gathers = []
    for ch in range(NCH):
        dst = pl.ds(ch * CHUNK, CHUNK)
        gathers.append(pltpu.async_copy(ent_hbm.at[idx_v.at[0, ch]], head_v.at[dst], s0))
        gathers.append(pltpu.async_copy(rel_hbm.at[idx_v.at[1, ch]], relv_v.at[dst], s1))
        gathers.append(pltpu.async_copy(ent_hbm.at[idx_v.at[2, ch]], tail_v.at[dst], s2))
    for g in gathers:
        g.wait()
    rows = pl.ds(base, BPW)
    w0 = pltpu.async_copy(head_v, out_hbm.at[rows, pl.ds(0, D)], s0)
    w1 = pltpu.async_copy(relv_v, out_hbm.at[rows, pl.ds(D, D)], s1)
    w2 = pltpu.async_copy(tail_v, out_hbm.at[rows, pl.ds(2 * D, D)], s2)
    w0.wait()
    w1.wait()
    w2.wait()


def kernel(positive_item, entity_embedding, relation_embedding):
    idx = positive_item.astype(jnp.int32)
    idx_arr = (idx.reshape(NW, NCH, CHUNK, 3)
                  .transpose(0, 3, 1, 2))
    out = _gather_kernel(idx_arr, entity_embedding, relation_embedding)
    return out.reshape(B, 3, 1, D)
